# baseline (speedup 1.0000x reference)
import math

import jax
import jax.numpy as jnp
import numpy as np
from jax import lax
from jax.experimental import pallas as pl
from jax.experimental.pallas import tpu as pltpu
from jax.experimental.pallas import tpu_sc as plsc

N_LEVELS = 16
F_PER = 2
LOG2_T = 19
T = 1 << LOG2_T
TW = T * F_PER
BASE = 16
MAXR = 2048
_growth = math.exp((math.log(MAXR) - math.log(BASE)) / (N_LEVELS - 1))
RES = [float(int(math.ceil(BASE * _growth ** l))) for l in range(N_LEVELS)]
OFFSETS = [(0, 0, 0), (0, 0, 1), (0, 1, 0), (0, 1, 1),
           (1, 0, 0), (1, 0, 1), (1, 1, 0), (1, 1, 1)]
P1 = np.uint32(2654435761).astype(np.int32)
P2 = np.int32(805459861)
MASK = np.int32(T - 1)

NC = 2
NS = 16
NW = NC * NS
LANES = 16

CHUNK = 512
CG = CHUNK // LANES
IDX_PER_CHUNK = CHUNK * F_PER * 8


def _body(pos_hbm, tab_hbm, res_hbm, out_hbm, shared, norm_v, res_v, idx_v,
          feats_v, out_lv, sem_pos, sem_gat, sem_out, sem_stage):
    sid = lax.axis_index("s")
    wid = sid * NC + lax.axis_index("c")
    m = pos_hbm.shape[0] // 3
    per_w = m // NW
    n_chunks = per_w // CHUNK
    base = wid * per_w

    hp = [pltpu.async_copy(pos_hbm.at[pl.ds(k * m + base, per_w)],
                           norm_v.at[pl.ds(k * per_w, per_w)], sem_pos)
          for k in range(3)]
    hp.append(pltpu.async_copy(res_hbm, res_v, sem_pos))
    for h in hp:
        h.wait()

    def norm_body(g, carry):
        o = g * LANES
        for k in range(3):
            p = norm_v[pl.ds(k * per_w + o, LANES)]
            norm_v[pl.ds(k * per_w + o, LANES)] = jnp.clip(
                (p + 1.0) * 0.5, 0.0, jnp.float32(1.0 - 1e-6))
        return carry

    lax.fori_loop(0, per_w // LANES, norm_body, 0)

    seg = TW // NS

    if True:
        def level_body(l, carry):
            pltpu.async_copy(tab_hbm.at[pl.ds(l * TW + sid * seg, seg)],
                             shared.at[pl.ds(sid * seg, seg)],
                             sem_stage).wait()
            plsc.subcore_barrier()

            r = res_v[pl.ds(l * LANES, LANES)]

            def chunk_body(k, carry2):
                cb = k * CHUNK
                fracs = []
                for g in range(CG):
                    o = cb + g * LANES
                    sx = norm_v[pl.ds(o, LANES)] * r
                    sy = norm_v[pl.ds(per_w + o, LANES)] * r
                    sz = norm_v[pl.ds(2 * per_w + o, LANES)] * r
                    x0 = sx.astype(jnp.int32)
                    y0 = sy.astype(jnp.int32)
                    z0 = sz.astype(jnp.int32)
                    fracs.append((sx - x0.astype(jnp.float32),
                                  sy - y0.astype(jnp.float32),
                                  sz - z0.astype(jnp.float32)))
                    hx = (x0, x0 + 1)
                    hy0 = y0 * P1
                    hy = (hy0, hy0 + P1)
                    hz0 = z0 * P2
                    hz = (hz0, hz0 + P2)
                    for c, (dx, dy, dz) in enumerate(OFFSETS):
                        e0 = ((hx[dx] ^ hy[dy] ^ hz[dz]) & MASK) * 2
                        idx_v[pl.ds((0 * 8 + c) * CHUNK + g * LANES, LANES)] = e0
                        idx_v[pl.ds((1 * 8 + c) * CHUNK + g * LANES, LANES)] = e0 + 1
                h1 = pltpu.async_copy(
                    shared.at[idx_v.at[pl.ds(0, IDX_PER_CHUNK // 2)]],
                    feats_v.at[pl.ds(0, IDX_PER_CHUNK // 2)], sem_gat)
                h2 = pltpu.async_copy(
                    shared.at[idx_v.at[pl.ds(IDX_PER_CHUNK // 2, IDX_PER_CHUNK // 2)]],
                    feats_v.at[pl.ds(IDX_PER_CHUNK // 2, IDX_PER_CHUNK // 2)], sem_gat)
                h1.wait()
                h2.wait()

                for g in range(CG):
                    fx, fy, fz = fracs[g]
                    omx = 1.0 - fx
                    omy = 1.0 - fy
                    omz = 1.0 - fz
                    for f in range(F_PER):
                        fb = f * 8 * CHUNK + g * LANES
                        v = [feats_v[pl.ds(fb + c * CHUNK, LANES)] for c in range(8)]
                        c00 = v[0] * omz + v[1] * fz
                        c01 = v[2] * omz + v[3] * fz
                        c10 = v[4] * omz + v[5] * fz
                        c11 = v[6] * omz + v[7] * fz
                        c0 = c00 * omy + c01 * fy
                        c1 = c10 * omy + c11 * fy
                        out_lv[pl.ds(f * CHUNK + g * LANES, LANES)] = \
                            c0 * omx + c1 * fx
                o1 = pltpu.async_copy(
                    out_lv.at[pl.ds(0, CHUNK)],
                    out_hbm.at[pl.ds((2 * l) * m + base + cb, CHUNK)], sem_out)
                o2 = pltpu.async_copy(
                    out_lv.at[pl.ds(CHUNK, CHUNK)],
                    out_hbm.at[pl.ds((2 * l + 1) * m + base + cb, CHUNK)], sem_out)
                o1.wait()
                o2.wait()
                return carry2

            lax.fori_loop(0, n_chunks, chunk_body, 0)
            plsc.subcore_barrier()
            return carry

        lax.fori_loop(0, N_LEVELS, level_body, 0)


def kernel(positions, hash_tables, chunk_size):
    m = positions.shape[0]
    pos_t = positions.T.reshape(-1) + 0.0
    tab = hash_tables.reshape(-1) + 0.0
    res_rep = jnp.asarray(np.repeat(np.asarray(RES, np.float32), LANES))

    run = pl.kernel(
        _body,
        out_type=jax.ShapeDtypeStruct((N_LEVELS * F_PER * m,), jnp.float32),
        mesh=plsc.VectorSubcoreMesh(core_axis_name="c", subcore_axis_name="s"),
        compiler_params=pltpu.CompilerParams(needs_layout_passes=False,
                                             use_tc_tiling_on_sc=False),
        scratch_types=[
            pltpu.VMEM_SHARED((TW,), jnp.float32),
            pltpu.VMEM((3 * (m // NW),), jnp.float32),
            pltpu.VMEM((N_LEVELS * LANES,), jnp.float32),
            pltpu.VMEM((IDX_PER_CHUNK,), jnp.int32),
            pltpu.VMEM((IDX_PER_CHUNK,), jnp.float32),
            pltpu.VMEM((F_PER * CHUNK,), jnp.float32),
            pltpu.SemaphoreType.DMA,
            pltpu.SemaphoreType.DMA,
            pltpu.SemaphoreType.DMA,
            pltpu.SemaphoreType.DMA,
        ],
    )
    out = run(pos_t, tab, res_rep)
    return out.reshape(N_LEVELS * F_PER, m).T


# speedup vs baseline: 17.8064x; 17.8064x over previous
import math

import jax
import jax.numpy as jnp
import numpy as np
from jax import lax
from jax.experimental import pallas as pl
from jax.experimental.pallas import tpu as pltpu
from jax.experimental.pallas import tpu_sc as plsc

N_LEVELS = 16
F_PER = 2
LOG2_T = 19
T = 1 << LOG2_T
TW = T * F_PER
BASE = 16
MAXR = 2048
_growth = math.exp((math.log(MAXR) - math.log(BASE)) / (N_LEVELS - 1))
RES = [float(int(math.ceil(BASE * _growth ** l))) for l in range(N_LEVELS)]
OFFSETS = [(0, 0, 0), (0, 0, 1), (0, 1, 0), (0, 1, 1),
           (1, 0, 0), (1, 0, 1), (1, 1, 0), (1, 1, 1)]
P1 = np.uint32(2654435761).astype(np.int32)
P2 = np.int32(805459861)
MASK = np.int32(T - 1)

NC = 2
NS = 16
NW = NC * NS
LANES = 16

CHUNK = 512
CG = CHUNK // LANES
IPC = CHUNK * F_PER * 8
FPC = 3 * CHUNK
OPC = F_PER * CHUNK


def _body(pos_hbm, tab_hbm, res_hbm, out_hbm, shared, norm_v, res_v, idx_v,
          feats_v, frac_v, out_lv, sem_pos, sem_stage, sg0, sg1, so0, so1):
    sid = lax.axis_index("s")
    wid = sid * NC + lax.axis_index("c")
    m = pos_hbm.shape[0] // 3
    per_w = m // NW
    n_chunks = per_w // CHUNK
    base = wid * per_w

    hp = [pltpu.async_copy(pos_hbm.at[pl.ds(k * m + base, per_w)],
                           norm_v.at[pl.ds(k * per_w, per_w)], sem_pos)
          for k in range(3)]
    hp.append(pltpu.async_copy(res_hbm, res_v, sem_pos))
    for h in hp:
        h.wait()

    def norm_body(g, carry):
        o = g * LANES
        for k in range(3):
            p = norm_v[pl.ds(k * per_w + o, LANES)]
            norm_v[pl.ds(k * per_w + o, LANES)] = jnp.clip(
                (p + 1.0) * 0.5, 0.0, jnp.float32(1.0 - 1e-6))
        return carry

    lax.fori_loop(0, per_w // LANES, norm_body, 0)

    seg = TW // NS

    def hash_chunk(k, r, s):
        cb = k * CHUNK
        gi = s * IPC
        gf = s * FPC
        for g in range(CG):
            o = cb + g * LANES
            sx = norm_v[pl.ds(o, LANES)] * r
            sy = norm_v[pl.ds(per_w + o, LANES)] * r
            sz = norm_v[pl.ds(2 * per_w + o, LANES)] * r
            x0 = sx.astype(jnp.int32)
            y0 = sy.astype(jnp.int32)
            z0 = sz.astype(jnp.int32)
            frac_v[pl.ds(gf + 0 * CHUNK + g * LANES, LANES)] = sx - x0.astype(jnp.float32)
            frac_v[pl.ds(gf + 1 * CHUNK + g * LANES, LANES)] = sy - y0.astype(jnp.float32)
            frac_v[pl.ds(gf + 2 * CHUNK + g * LANES, LANES)] = sz - z0.astype(jnp.float32)
            hx = (x0, x0 + 1)
            hy0 = y0 * P1
            hy = (hy0, hy0 + P1)
            hz0 = z0 * P2
            hz = (hz0, hz0 + P2)
            for c, (dx, dy, dz) in enumerate(OFFSETS):
                h = (hx[dx] ^ hy[dy] ^ hz[dz]) & MASK
                e0 = h + (h & jnp.int32(-128))
                idx_v[pl.ds(gi + (0 * 8 + c) * CHUNK + g * LANES, LANES)] = e0
                idx_v[pl.ds(gi + (1 * 8 + c) * CHUNK + g * LANES, LANES)] = \
                    e0 + jnp.int32(128)

        for slot, sem in ((0, sg0), (1, sg1)):
            @pl.when(s == slot)
            def _():
                pltpu.async_copy(
                    shared.at[idx_v.at[pl.ds(gi, IPC // 2)]],
                    feats_v.at[pl.ds(gi, IPC // 2)], sem)
                pltpu.async_copy(
                    shared.at[idx_v.at[pl.ds(gi + IPC // 2, IPC // 2)]],
                    feats_v.at[pl.ds(gi + IPC // 2, IPC // 2)], sem)

    def interp_chunk(l, k, s):
        cb = k * CHUNK
        gi = s * IPC
        gf = s * FPC
        go = s * OPC
        for slot, sem in ((0, sg0), (1, sg1)):
            @pl.when(s == slot)
            def _():
                for half in range(2):
                    pltpu.make_async_copy(
                        shared.at[idx_v.at[pl.ds(gi + half * (IPC // 2), IPC // 2)]],
                        feats_v.at[pl.ds(gi + half * (IPC // 2), IPC // 2)],
                        sem).wait()
        @pl.when(k >= 2)
        def _():
            for slot, sem in ((0, so0), (1, so1)):
                @pl.when(s == slot)
                def _():
                    for f in range(F_PER):
                        pltpu.make_async_copy(
                            out_lv.at[pl.ds(go + f * CHUNK, CHUNK)],
                            out_hbm.at[pl.ds(base, CHUNK)], sem).wait()

        for g in range(CG):
            fx = frac_v[pl.ds(gf + 0 * CHUNK + g * LANES, LANES)]
            fy = frac_v[pl.ds(gf + 1 * CHUNK + g * LANES, LANES)]
            fz = frac_v[pl.ds(gf + 2 * CHUNK + g * LANES, LANES)]
            omx = 1.0 - fx
            omy = 1.0 - fy
            omz = 1.0 - fz
            for f in range(F_PER):
                fb = gi + f * 8 * CHUNK + g * LANES
                v = [feats_v[pl.ds(fb + c * CHUNK, LANES)] for c in range(8)]
                c00 = v[0] * omz + v[1] * fz
                c01 = v[2] * omz + v[3] * fz
                c10 = v[4] * omz + v[5] * fz
                c11 = v[6] * omz + v[7] * fz
                c0 = c00 * omy + c01 * fy
                c1 = c10 * omy + c11 * fy
                out_lv[pl.ds(go + f * CHUNK + g * LANES, LANES)] = \
                    c0 * omx + c1 * fx
        for slot, sem in ((0, so0), (1, so1)):
            @pl.when(s == slot)
            def _():
                for f in range(F_PER):
                    pltpu.async_copy(
                        out_lv.at[pl.ds(go + f * CHUNK, CHUNK)],
                        out_hbm.at[pl.ds((2 * l + f) * m + base + cb, CHUNK)],
                        sem)

    def level_body(l, carry):
        pltpu.async_copy(tab_hbm.at[pl.ds(l * TW + sid * seg, seg)],
                         shared.at[pl.ds(sid * seg, seg)],
                         sem_stage).wait()
        plsc.subcore_barrier()

        r = res_v[pl.ds(l * LANES, LANES)]

        def pipe_body(k, carry2):
            @pl.when(k < n_chunks)
            def _():
                hash_chunk(k, r, lax.rem(k, 2))

            @pl.when(k >= 1)
            def _():
                interp_chunk(l, k - 1, lax.rem(k - 1, 2))

            return carry2

        lax.fori_loop(0, n_chunks + 1, pipe_body, 0)

        for sem in (so0, so1):
            for f in range(F_PER):
                pltpu.make_async_copy(
                    out_lv.at[pl.ds(f * CHUNK, CHUNK)],
                    out_hbm.at[pl.ds(base, CHUNK)], sem).wait()
        plsc.subcore_barrier()
        return carry

    lax.fori_loop(0, N_LEVELS, level_body, 0)


def kernel(positions, hash_tables, chunk_size):
    m = positions.shape[0]
    pos_t = positions.T.reshape(-1)
    tab = hash_tables.reshape(N_LEVELS, T // 128, 128, F_PER) \
                     .transpose(0, 1, 3, 2).reshape(-1)
    res_rep = jnp.asarray(np.repeat(np.asarray(RES, np.float32), LANES))

    run = pl.kernel(
        _body,
        out_type=jax.ShapeDtypeStruct((N_LEVELS * F_PER * m,), jnp.float32),
        mesh=plsc.VectorSubcoreMesh(core_axis_name="c", subcore_axis_name="s"),
        compiler_params=pltpu.CompilerParams(needs_layout_passes=False,
                                             use_tc_tiling_on_sc=False),
        scratch_types=[
            pltpu.VMEM_SHARED((TW,), jnp.float32),
            pltpu.VMEM((3 * (m // NW),), jnp.float32),
            pltpu.VMEM((N_LEVELS * LANES,), jnp.float32),
            pltpu.VMEM((2 * IPC,), jnp.int32),
            pltpu.VMEM((2 * IPC,), jnp.float32),
            pltpu.VMEM((2 * FPC,), jnp.float32),
            pltpu.VMEM((2 * OPC,), jnp.float32),
            pltpu.SemaphoreType.DMA,
            pltpu.SemaphoreType.DMA,
            pltpu.SemaphoreType.DMA,
            pltpu.SemaphoreType.DMA,
            pltpu.SemaphoreType.DMA,
            pltpu.SemaphoreType.DMA,
        ],
    )
    out = run(pos_t, tab, res_rep)
    return out.reshape(N_LEVELS * F_PER, m).T


# speedup vs baseline: 22.4987x; 1.2635x over previous
import math

import jax
import jax.numpy as jnp
import numpy as np
from jax import lax
from jax.experimental import pallas as pl
from jax.experimental.pallas import tpu as pltpu
from jax.experimental.pallas import tpu_sc as plsc

N_LEVELS = 16
F_PER = 2
LOG2_T = 19
T = 1 << LOG2_T
BASE = 16
MAXR = 2048
_growth = math.exp((math.log(MAXR) - math.log(BASE)) / (N_LEVELS - 1))
RES = [float(int(math.ceil(BASE * _growth ** l))) for l in range(N_LEVELS)]
OFFSETS = [(0, 0, 0), (0, 0, 1), (0, 1, 0), (0, 1, 1),
           (1, 0, 0), (1, 0, 1), (1, 1, 0), (1, 1, 1)]
P1 = np.uint32(2654435761).astype(np.int32)
P2 = np.int32(805459861)
MASK = np.int32(T - 1)

NC = 2
NS = 16
NW = NC * NS
LANES = 16

CHUNK = 512
CG = CHUNK // LANES
IPC = CHUNK * 8
FPC = 3 * CHUNK
OPC = F_PER * CHUNK

HI_MASK = np.int32(-65536)


def _body(pos_hbm, tab_hbm, res_hbm, out_hbm, shared, norm_v, res_v, idx_v,
          feats_v, frac_v, out_lv, sem_pos, sem_stage, sg0, sg1, so0, so1):
    sid = lax.axis_index("s")
    wid = sid * NC + lax.axis_index("c")
    m = pos_hbm.shape[0] // 3
    per_w = m // NW
    n_chunks = per_w // CHUNK
    base = wid * per_w

    hp = [pltpu.async_copy(pos_hbm.at[pl.ds(k * m + base, per_w)],
                           norm_v.at[pl.ds(k * per_w, per_w)], sem_pos)
          for k in range(3)]
    hp.append(pltpu.async_copy(res_hbm, res_v, sem_pos))
    for h in hp:
        h.wait()

    def norm_body(g, carry):
        o = g * LANES
        for k in range(3):
            p = norm_v[pl.ds(k * per_w + o, LANES)]
            norm_v[pl.ds(k * per_w + o, LANES)] = jnp.clip(
                (p + 1.0) * 0.5, 0.0, jnp.float32(1.0 - 1e-6))
        return carry

    lax.fori_loop(0, per_w // LANES, norm_body, 0)

    seg = T // NS

    def hash_chunk(k, r, s):
        cb = k * CHUNK
        gi = s * IPC
        gf = s * FPC
        for g in range(CG):
            o = cb + g * LANES
            sx = norm_v[pl.ds(o, LANES)] * r
            sy = norm_v[pl.ds(per_w + o, LANES)] * r
            sz = norm_v[pl.ds(2 * per_w + o, LANES)] * r
            x0 = sx.astype(jnp.int32)
            y0 = sy.astype(jnp.int32)
            z0 = sz.astype(jnp.int32)
            frac_v[pl.ds(gf + 0 * CHUNK + g * LANES, LANES)] = sx - x0.astype(jnp.float32)
            frac_v[pl.ds(gf + 1 * CHUNK + g * LANES, LANES)] = sy - y0.astype(jnp.float32)
            frac_v[pl.ds(gf + 2 * CHUNK + g * LANES, LANES)] = sz - z0.astype(jnp.float32)
            hx = (x0, x0 + 1)
            hy0 = y0 * P1
            hy = (hy0, hy0 + P1)
            hz0 = z0 * P2
            hz = (hz0, hz0 + P2)
            for c, (dx, dy, dz) in enumerate(OFFSETS):
                idx_v[pl.ds(gi + c * CHUNK + g * LANES, LANES)] = \
                    (hx[dx] ^ hy[dy] ^ hz[dz]) & MASK

        for slot, sem in ((0, sg0), (1, sg1)):
            @pl.when(s == slot)
            def _():
                pltpu.async_copy(
                    shared.at[idx_v.at[pl.ds(gi, IPC)]],
                    feats_v.at[pl.ds(gi, IPC)], sem)

    def interp_chunk(l, k, s):
        cb = k * CHUNK
        gi = s * IPC
        gf = s * FPC
        go = s * OPC
        for slot, sem in ((0, sg0), (1, sg1)):
            @pl.when(s == slot)
            def _():
                pltpu.make_async_copy(
                    shared.at[idx_v.at[pl.ds(gi, IPC)]],
                    feats_v.at[pl.ds(gi, IPC)], sem).wait()
        @pl.when(k >= 2)
        def _():
            for slot, sem in ((0, so0), (1, so1)):
                @pl.when(s == slot)
                def _():
                    for f in range(F_PER):
                        pltpu.make_async_copy(
                            out_lv.at[pl.ds(go + f * CHUNK, CHUNK)],
                            out_hbm.at[pl.ds(base, CHUNK)], sem).wait()

        for g in range(CG):
            fx = frac_v[pl.ds(gf + 0 * CHUNK + g * LANES, LANES)]
            fy = frac_v[pl.ds(gf + 1 * CHUNK + g * LANES, LANES)]
            fz = frac_v[pl.ds(gf + 2 * CHUNK + g * LANES, LANES)]
            omx = 1.0 - fx
            omy = 1.0 - fy
            omz = 1.0 - fz
            v0 = []
            v1 = []
            for c in range(8):
                w = feats_v[pl.ds(gi + c * CHUNK + g * LANES, LANES)]
                v0.append(plsc.bitcast(w & HI_MASK, jnp.float32))
                v1.append(plsc.bitcast(w << 16, jnp.float32))
            for f, v in ((0, v0), (1, v1)):
                c00 = v[0] * omz + v[1] * fz
                c01 = v[2] * omz + v[3] * fz
                c10 = v[4] * omz + v[5] * fz
                c11 = v[6] * omz + v[7] * fz
                c0 = c00 * omy + c01 * fy
                c1 = c10 * omy + c11 * fy
                out_lv[pl.ds(go + f * CHUNK + g * LANES, LANES)] = \
                    c0 * omx + c1 * fx
        for slot, sem in ((0, so0), (1, so1)):
            @pl.when(s == slot)
            def _():
                for f in range(F_PER):
                    pltpu.async_copy(
                        out_lv.at[pl.ds(go + f * CHUNK, CHUNK)],
                        out_hbm.at[pl.ds((2 * l + f) * m + base + cb, CHUNK)],
                        sem)

    def level_body(l, carry):
        pltpu.async_copy(tab_hbm.at[pl.ds(l * T + sid * seg, seg)],
                         shared.at[pl.ds(sid * seg, seg)],
                         sem_stage).wait()
        plsc.subcore_barrier()

        r = res_v[pl.ds(l * LANES, LANES)]

        def pipe_body(k, carry2):
            @pl.when(k < n_chunks)
            def _():
                hash_chunk(k, r, lax.rem(k, 2))

            @pl.when(k >= 1)
            def _():
                interp_chunk(l, k - 1, lax.rem(k - 1, 2))

            return carry2

        lax.fori_loop(0, n_chunks + 1, pipe_body, 0)

        for sem in (so0, so1):
            for f in range(F_PER):
                pltpu.make_async_copy(
                    out_lv.at[pl.ds(f * CHUNK, CHUNK)],
                    out_hbm.at[pl.ds(base, CHUNK)], sem).wait()
        plsc.subcore_barrier()
        return carry

    lax.fori_loop(0, N_LEVELS, level_body, 0)


def kernel(positions, hash_tables, chunk_size):
    m = positions.shape[0]
    pos_t = positions.T.reshape(-1)
    u = lax.bitcast_convert_type(hash_tables.astype(jnp.bfloat16),
                                 jnp.uint16).astype(jnp.uint32)
    packed = ((u[..., 0] << 16) | u[..., 1]).astype(jnp.int32).reshape(-1)
    res_rep = jnp.asarray(np.repeat(np.asarray(RES, np.float32), LANES))

    run = pl.kernel(
        _body,
        out_type=jax.ShapeDtypeStruct((N_LEVELS * F_PER * m,), jnp.float32),
        mesh=plsc.VectorSubcoreMesh(core_axis_name="c", subcore_axis_name="s"),
        compiler_params=pltpu.CompilerParams(needs_layout_passes=False,
                                             use_tc_tiling_on_sc=False),
        scratch_types=[
            pltpu.VMEM_SHARED((T,), jnp.int32),
            pltpu.VMEM((3 * (m // NW),), jnp.float32),
            pltpu.VMEM((N_LEVELS * LANES,), jnp.float32),
            pltpu.VMEM((2 * IPC,), jnp.int32),
            pltpu.VMEM((2 * IPC,), jnp.int32),
            pltpu.VMEM((2 * FPC,), jnp.float32),
            pltpu.VMEM((2 * OPC,), jnp.float32),
            pltpu.SemaphoreType.DMA,
            pltpu.SemaphoreType.DMA,
            pltpu.SemaphoreType.DMA,
            pltpu.SemaphoreType.DMA,
            pltpu.SemaphoreType.DMA,
            pltpu.SemaphoreType.DMA,
        ],
    )
    out = run(pos_t, packed, res_rep)
    return out.reshape(N_LEVELS * F_PER, m).T


# speedup vs baseline: 23.7645x; 1.0563x over previous
import math

import jax
import jax.numpy as jnp
import numpy as np
from jax import lax
from jax.experimental import pallas as pl
from jax.experimental.pallas import tpu as pltpu
from jax.experimental.pallas import tpu_sc as plsc

N_LEVELS = 16
F_PER = 2
LOG2_T = 19
T = 1 << LOG2_T
BASE = 16
MAXR = 2048
_growth = math.exp((math.log(MAXR) - math.log(BASE)) / (N_LEVELS - 1))
RES = [float(int(math.ceil(BASE * _growth ** l))) for l in range(N_LEVELS)]
OFFSETS = [(0, 0, 0), (0, 0, 1), (0, 1, 0), (0, 1, 1),
           (1, 0, 0), (1, 0, 1), (1, 1, 0), (1, 1, 1)]
P1 = np.uint32(2654435761).astype(np.int32)
P2 = np.int32(805459861)
MASK = np.int32(T - 1)

NC = 2
NS = 16
NW = NC * NS
LANES = 16

CHUNK = 512
CG = CHUNK // LANES
IPC = CHUNK * 8
FPC = 3 * CHUNK
OPC = F_PER * CHUNK

HI_MASK = np.int32(-65536)


def _body(pos_hbm, tab_hbm, res_hbm, out_hbm, shared, norm_v, res_v, idx_v,
          feats_v, frac_v, out_lv, sem_pos, sem_stage, sg0, sg1, so0, so1):
    sid = lax.axis_index("s")
    wid = sid * NC + lax.axis_index("c")
    m = pos_hbm.shape[0] // 3
    per_w = m // NW
    n_chunks = per_w // CHUNK
    base = wid * per_w

    hp = [pltpu.async_copy(pos_hbm.at[pl.ds(k * m + base, per_w)],
                           norm_v.at[pl.ds(k * per_w, per_w)], sem_pos)
          for k in range(3)]
    hp.append(pltpu.async_copy(res_hbm, res_v, sem_pos))
    for h in hp:
        h.wait()

    def norm_body(g, carry):
        o = g * LANES
        for k in range(3):
            p = norm_v[pl.ds(k * per_w + o, LANES)]
            norm_v[pl.ds(k * per_w + o, LANES)] = jnp.clip(
                (p + 1.0) * 0.5, 0.0, jnp.float32(1.0 - 1e-6))
        return carry

    lax.fori_loop(0, per_w // LANES, norm_body, 0)

    seg = T // NS

    def hash_chunk(k, r, s, pofs):
        cb = k * CHUNK
        gi = s * IPC
        gf = s * FPC
        for g in range(CG):
            o = cb + g * LANES
            sx = norm_v[pl.ds(o, LANES)] * r
            sy = norm_v[pl.ds(per_w + o, LANES)] * r
            sz = norm_v[pl.ds(2 * per_w + o, LANES)] * r
            x0 = sx.astype(jnp.int32)
            y0 = sy.astype(jnp.int32)
            z0 = sz.astype(jnp.int32)
            frac_v[pl.ds(gf + 0 * CHUNK + g * LANES, LANES)] = sx - x0.astype(jnp.float32)
            frac_v[pl.ds(gf + 1 * CHUNK + g * LANES, LANES)] = sy - y0.astype(jnp.float32)
            frac_v[pl.ds(gf + 2 * CHUNK + g * LANES, LANES)] = sz - z0.astype(jnp.float32)
            hx = (x0, x0 + 1)
            hy0 = y0 * P1
            hy = (hy0, hy0 + P1)
            hz0 = z0 * P2
            hz = (hz0, hz0 + P2)
            for c, (dx, dy, dz) in enumerate(OFFSETS):
                idx_v[pl.ds(gi + c * CHUNK + g * LANES, LANES)] = \
                    ((hx[dx] ^ hy[dy] ^ hz[dz]) & MASK) + pofs

        for slot, sem in ((0, sg0), (1, sg1)):
            @pl.when(s == slot)
            def _():
                pltpu.async_copy(
                    shared.at[idx_v.at[pl.ds(gi, IPC)]],
                    feats_v.at[pl.ds(gi, IPC)], sem)

    def interp_chunk(l, k, s):
        cb = k * CHUNK
        gi = s * IPC
        gf = s * FPC
        go = s * OPC
        for slot, sem in ((0, sg0), (1, sg1)):
            @pl.when(s == slot)
            def _():
                pltpu.make_async_copy(
                    shared.at[idx_v.at[pl.ds(gi, IPC)]],
                    feats_v.at[pl.ds(gi, IPC)], sem).wait()
        @pl.when(k >= 2)
        def _():
            for slot, sem in ((0, so0), (1, so1)):
                @pl.when(s == slot)
                def _():
                    for f in range(F_PER):
                        pltpu.make_async_copy(
                            out_lv.at[pl.ds(go + f * CHUNK, CHUNK)],
                            out_hbm.at[pl.ds(base, CHUNK)], sem).wait()

        for g in range(CG):
            fx = frac_v[pl.ds(gf + 0 * CHUNK + g * LANES, LANES)]
            fy = frac_v[pl.ds(gf + 1 * CHUNK + g * LANES, LANES)]
            fz = frac_v[pl.ds(gf + 2 * CHUNK + g * LANES, LANES)]
            omx = 1.0 - fx
            omy = 1.0 - fy
            omz = 1.0 - fz
            v0 = []
            v1 = []
            for c in range(8):
                w = feats_v[pl.ds(gi + c * CHUNK + g * LANES, LANES)]
                v0.append(plsc.bitcast(w & HI_MASK, jnp.float32))
                v1.append(plsc.bitcast(w << 16, jnp.float32))
            for f, v in ((0, v0), (1, v1)):
                c00 = v[0] * omz + v[1] * fz
                c01 = v[2] * omz + v[3] * fz
                c10 = v[4] * omz + v[5] * fz
                c11 = v[6] * omz + v[7] * fz
                c0 = c00 * omy + c01 * fy
                c1 = c10 * omy + c11 * fy
                out_lv[pl.ds(go + f * CHUNK + g * LANES, LANES)] = \
                    c0 * omx + c1 * fx
        for slot, sem in ((0, so0), (1, so1)):
            @pl.when(s == slot)
            def _():
                for f in range(F_PER):
                    pltpu.async_copy(
                        out_lv.at[pl.ds(go + f * CHUNK, CHUNK)],
                        out_hbm.at[pl.ds((2 * l + f) * m + base + cb, CHUNK)],
                        sem)

    pltpu.async_copy(tab_hbm.at[pl.ds(sid * seg, seg)],
                     shared.at[pl.ds(sid * seg, seg)], sem_stage)

    def level_body(l, carry):
        p = lax.rem(l, 2)
        pofs = p * jnp.int32(T)
        pltpu.make_async_copy(tab_hbm.at[pl.ds(l * T + sid * seg, seg)],
                              shared.at[pl.ds(p * T + sid * seg, seg)],
                              sem_stage).wait()
        plsc.subcore_barrier()

        @pl.when(l + 1 < N_LEVELS)
        def _():
            pltpu.async_copy(
                tab_hbm.at[pl.ds((l + 1) * T + sid * seg, seg)],
                shared.at[pl.ds((1 - p) * T + sid * seg, seg)], sem_stage)

        r = res_v[pl.ds(l * LANES, LANES)]

        def pipe_body(k, carry2):
            @pl.when(k < n_chunks)
            def _():
                hash_chunk(k, r, lax.rem(k, 2), pofs)

            @pl.when(k >= 1)
            def _():
                interp_chunk(l, k - 1, lax.rem(k - 1, 2))

            return carry2

        lax.fori_loop(0, n_chunks + 1, pipe_body, 0)

        for sem in (so0, so1):
            for f in range(F_PER):
                pltpu.make_async_copy(
                    out_lv.at[pl.ds(f * CHUNK, CHUNK)],
                    out_hbm.at[pl.ds(base, CHUNK)], sem).wait()
        return carry

    lax.fori_loop(0, N_LEVELS, level_body, 0)


def kernel(positions, hash_tables, chunk_size):
    m = positions.shape[0]
    pos_t = positions.T.reshape(-1)
    u = lax.bitcast_convert_type(hash_tables.astype(jnp.bfloat16),
                                 jnp.uint16).astype(jnp.uint32)
    packed = ((u[..., 0] << 16) | u[..., 1]).astype(jnp.int32).reshape(-1)
    res_rep = jnp.asarray(np.repeat(np.asarray(RES, np.float32), LANES))

    run = pl.kernel(
        _body,
        out_type=jax.ShapeDtypeStruct((N_LEVELS * F_PER * m,), jnp.float32),
        mesh=plsc.VectorSubcoreMesh(core_axis_name="c", subcore_axis_name="s"),
        compiler_params=pltpu.CompilerParams(needs_layout_passes=False,
                                             use_tc_tiling_on_sc=False),
        scratch_types=[
            pltpu.VMEM_SHARED((2 * T,), jnp.int32),
            pltpu.VMEM((3 * (m // NW),), jnp.float32),
            pltpu.VMEM((N_LEVELS * LANES,), jnp.float32),
            pltpu.VMEM((2 * IPC,), jnp.int32),
            pltpu.VMEM((2 * IPC,), jnp.int32),
            pltpu.VMEM((2 * FPC,), jnp.float32),
            pltpu.VMEM((2 * OPC,), jnp.float32),
            pltpu.SemaphoreType.DMA,
            pltpu.SemaphoreType.DMA,
            pltpu.SemaphoreType.DMA,
            pltpu.SemaphoreType.DMA,
            pltpu.SemaphoreType.DMA,
            pltpu.SemaphoreType.DMA,
        ],
    )
    out = run(pos_t, packed, res_rep)
    return out.reshape(N_LEVELS * F_PER, m).T
